# baseline (speedup 1.0000x reference)
import functools

import jax
import jax.numpy as jnp
from jax import lax
from jax.experimental import pallas as pl
from jax.experimental.pallas import tpu as pltpu
from jax.experimental.pallas import tpu_sc as plsc

_LANES = 16


_GCHUNK = 128


def _build_lookup(B, F, V, D, n_workers):
    rows_per_w = (B // n_workers) * F
    out_per_w = (B // n_workers) * D
    n_chunks = out_per_w // _LANES
    g_chunks = rows_per_w // _GCHUNK
    mesh = plsc.VectorSubcoreMesh(core_axis_name="c", subcore_axis_name="s")
    num_cores = plsc.get_sparse_core_info().num_cores

    @functools.partial(
        pl.kernel,
        mesh=mesh,
        compiler_params=pltpu.CompilerParams(
            use_tc_tiling_on_sc=False, needs_layout_passes=False),
        out_type=jax.ShapeDtypeStruct((B * D,), jnp.float32),
        scratch_types=[
            pltpu.VMEM((g_chunks, _GCHUNK), jnp.int32),
            pltpu.VMEM((rows_per_w, D), jnp.float32),
            pltpu.VMEM((out_per_w,), jnp.float32),
            pltpu.VMEM((D * _LANES,), jnp.float32),
            pltpu.SemaphoreType.DMA,
        ],
    )
    def lookup(idx_hbm, table_hbm, bias_hbm, out_hbm,
               idx_v, rows_v, out_v, bias_v, sem):
        wid = lax.axis_index("s") * num_cores + lax.axis_index("c")
        pltpu.sync_copy(bias_hbm, bias_v)
        pltpu.sync_copy(idx_hbm.at[pl.ds(wid * g_chunks, g_chunks)], idx_v)

        def fire(j, c):
            pltpu.async_copy(table_hbm.at[idx_v.at[j]],
                             rows_v.at[pl.ds(j * _GCHUNK, _GCHUNK)], sem).wait()
            return c

        lax.fori_loop(0, g_chunks, fire, 0)

        lane = lax.iota(jnp.int32, _LANES)
        step = jnp.full((_LANES,), _LANES, jnp.int32)
        dvec = jnp.full((_LANES,), D, jnp.int32)
        fvec = jnp.full((_LANES,), F, jnp.int32)

        def chunk_body(i, flat):
            b_local = lax.div(flat, dvec)
            d = flat - b_local * dvec
            row0 = b_local * fvec
            acc = bias_v[pl.ds((i % D) * _LANES, _LANES)]
            for f in range(F):
                acc = acc + plsc.load_gather(rows_v, [row0 + f, d])
            out_v[pl.ds(i * _LANES, _LANES)] = acc
            return flat + step

        lax.fori_loop(0, n_chunks, chunk_body, lane)
        pltpu.sync_copy(out_v, out_hbm.at[pl.ds(wid * out_per_w, out_per_w)])

    return lookup


def kernel(x, table, bias):
    B, F = x.shape
    V, D = table.shape
    info = plsc.get_sparse_core_info()
    n_workers = info.num_cores * info.num_subcores

    idx_flat = x.reshape(-1).astype(jnp.int32).reshape(-1, _GCHUNK)
    phase = (jnp.arange(D * _LANES, dtype=jnp.int32)) % D
    bias_t = bias[phase]

    out_flat = _build_lookup(B, F, V, D, n_workers)(idx_flat, table, bias_t)
    return out_flat.reshape(B, D)
